# baseline (speedup 1.0000x reference)
import jax
import jax.numpy as jnp
from jax import lax
from jax.experimental import pallas as pl
from jax.experimental.pallas import tpu as pltpu
from jax.experimental.pallas import tpu_sc as plsc

EMB = 16
BATCH = 16384
ROWS_PACKED = 125000
PACK = 128

NUM_WORKERS = 32
B_PER_W = BATCH // NUM_WORKERS
LANES = 16
CHUNK = 128
N_CHUNKS = B_PER_W // CHUNK
GROUPS_PER_CHUNK = CHUNK // LANES


def _mf_body(user_hbm, item_hbm, ut_hbm, it_hbm, out_hbm,
             uidx_vm, iidx_vm, urow_vm, irow_vm,
             ubuf0, ubuf1, ibuf0, ibuf1, out_v, sem0, sem1):
    wid = lax.axis_index("s") * 2 + lax.axis_index("c")
    base = wid * B_PER_W

    pltpu.sync_copy(user_hbm.at[pl.ds(base, B_PER_W)], uidx_vm)
    pltpu.sync_copy(item_hbm.at[pl.ds(base, B_PER_W)], iidx_vm)

    def rows_body(j, _):
        off = pl.multiple_of(j * LANES, LANES)
        urow_vm[pl.ds(off, LANES)] = uidx_vm[pl.ds(off, LANES)] >> 3
        irow_vm[pl.ds(off, LANES)] = iidx_vm[pl.ds(off, LANES)] >> 3
        return ()

    lax.fori_loop(0, B_PER_W // LANES, rows_body, ())

    ubufs = (ubuf0, ubuf1)
    ibufs = (ibuf0, ibuf1)
    sems = (sem0, sem1)

    def fire(c, par):
        sl = pl.ds(pl.multiple_of(c * CHUNK, CHUNK), CHUNK)
        pltpu.async_copy(ut_hbm.at[urow_vm.at[sl]], ubufs[par], sems[par])
        pltpu.async_copy(it_hbm.at[irow_vm.at[sl]], ibufs[par], sems[par])

    def drain_and_compute(c, par):
        ub, ib, sem = ubufs[par], ibufs[par], sems[par]
        pltpu.make_async_copy(ut_hbm.at[pl.ds(0, CHUNK)], ub, sem).wait()
        pltpu.make_async_copy(it_hbm.at[pl.ds(0, CHUNK)], ib, sem).wait()
        coff = pl.multiple_of(c * CHUNK, CHUNK)
        lanes = lax.iota(jnp.int32, LANES)
        for j in range(GROUPS_PER_CHUNK):
            boff = pl.multiple_of(coff + j * LANES, LANES)
            p_vec = j * LANES + lanes
            usub = (uidx_vm[pl.ds(boff, LANES)] & 7) * EMB
            isub = (iidx_vm[pl.ds(boff, LANES)] & 7) * EMB
            acc = jnp.zeros((LANES,), jnp.float32)
            for e in range(EMB):
                uv = plsc.load_gather(ub, [p_vec, usub + e])
                iv = plsc.load_gather(ib, [p_vec, isub + e])
                acc = acc + uv * iv
            out_v[pl.ds(boff, LANES)] = acc

    fire(0, 0)
    fire(1, 1)
    drain_and_compute(0, 0)
    fire(2, 0)
    drain_and_compute(1, 1)
    fire(3, 1)
    drain_and_compute(2, 0)
    drain_and_compute(3, 1)

    pltpu.sync_copy(out_v, out_hbm.at[pl.ds(base, B_PER_W)])


@jax.jit
def _mf(user, item, ut, it):
    mesh = plsc.VectorSubcoreMesh(core_axis_name="c", subcore_axis_name="s")
    f = pl.kernel(
        _mf_body,
        mesh=mesh,
        compiler_params=pltpu.CompilerParams(
            needs_layout_passes=False, use_tc_tiling_on_sc=False),
        out_type=jax.ShapeDtypeStruct((BATCH,), jnp.float32),
        scratch_types=[
            pltpu.VMEM((B_PER_W,), jnp.int32),
            pltpu.VMEM((B_PER_W,), jnp.int32),
            pltpu.VMEM((B_PER_W,), jnp.int32),
            pltpu.VMEM((B_PER_W,), jnp.int32),
            pltpu.VMEM((CHUNK, PACK), jnp.float32),
            pltpu.VMEM((CHUNK, PACK), jnp.float32),
            pltpu.VMEM((CHUNK, PACK), jnp.float32),
            pltpu.VMEM((CHUNK, PACK), jnp.float32),
            pltpu.VMEM((B_PER_W,), jnp.float32),
            pltpu.SemaphoreType.DMA,
            pltpu.SemaphoreType.DMA,
        ],
    )
    return f(user, item, ut, it)


def kernel(user, item, embed_user_GMF, embed_item_GMF):
    user = user.astype(jnp.int32)
    item = item.astype(jnp.int32)
    ut = embed_user_GMF.reshape(ROWS_PACKED, PACK)
    it = embed_item_GMF.reshape(ROWS_PACKED, PACK)
    return _mf(user, item, ut, it)


# speedup vs baseline: 1.0012x; 1.0012x over previous
import jax
import jax.numpy as jnp
from jax import lax
from jax.experimental import pallas as pl
from jax.experimental.pallas import tpu as pltpu
from jax.experimental.pallas import tpu_sc as plsc

EMB = 16
BATCH = 16384

NUM_WORKERS = 32
B_PER_W = BATCH // NUM_WORKERS
GATHER_CHUNK = 128
N_CHUNKS = B_PER_W // GATHER_CHUNK
LANES = 16
N_GROUPS = B_PER_W // LANES


def _mf_body(user_hbm, item_hbm, utab_hbm, itab_hbm, out_hbm,
             uidx_v, iidx_v, urows_v, irows_v, out_v, sem):
    wid = lax.axis_index("s") * 2 + lax.axis_index("c")
    base = wid * B_PER_W

    pltpu.sync_copy(user_hbm.at[pl.ds(base, B_PER_W)], uidx_v)
    pltpu.sync_copy(item_hbm.at[pl.ds(base, B_PER_W)], iidx_v)

    copies = []
    for c in range(N_CHUNKS):
        sl = pl.ds(c * GATHER_CHUNK, GATHER_CHUNK)
        copies.append(pltpu.async_copy(
            utab_hbm.at[uidx_v.at[sl]], urows_v.at[sl], sem))
        copies.append(pltpu.async_copy(
            itab_hbm.at[iidx_v.at[sl]], irows_v.at[sl], sem))
    for cp in copies:
        cp.wait()

    lanes = lax.iota(jnp.int32, LANES)

    def group_body(g, _):
        rows = g * LANES + lanes
        acc = jnp.zeros((LANES,), jnp.float32)
        for e in range(EMB):
            cols = jnp.full((LANES,), e, jnp.int32)
            uv = plsc.load_gather(urows_v, [rows, cols])
            iv = plsc.load_gather(irows_v, [rows, cols])
            acc = acc + uv * iv
        out_v[pl.ds(g * LANES, LANES)] = acc
        return ()

    lax.fori_loop(0, N_GROUPS, group_body, ())

    pltpu.sync_copy(out_v, out_hbm.at[pl.ds(base, B_PER_W)])


@jax.jit
def _mf(user, item, embed_user_GMF, embed_item_GMF):
    mesh = plsc.VectorSubcoreMesh(core_axis_name="c", subcore_axis_name="s")
    f = pl.kernel(
        _mf_body,
        mesh=mesh,
        compiler_params=pltpu.CompilerParams(
            needs_layout_passes=False, use_tc_tiling_on_sc=False),
        out_type=jax.ShapeDtypeStruct((BATCH,), jnp.float32),
        scratch_types=[
            pltpu.VMEM((B_PER_W,), jnp.int32),
            pltpu.VMEM((B_PER_W,), jnp.int32),
            pltpu.VMEM((B_PER_W, EMB), jnp.float32),
            pltpu.VMEM((B_PER_W, EMB), jnp.float32),
            pltpu.VMEM((B_PER_W,), jnp.float32),
            pltpu.SemaphoreType.DMA,
        ],
    )
    return f(user, item, embed_user_GMF, embed_item_GMF)


def kernel(user, item, embed_user_GMF, embed_item_GMF):
    user = user.astype(jnp.int32)
    item = item.astype(jnp.int32)
    return _mf(user, item, embed_user_GMF, embed_item_GMF)
